# Initial kernel scaffold; baseline (speedup 1.0000x reference)
import jax
import jax.numpy as jnp
from jax.experimental import pallas as pl


def kernel(x, edge_index, edge_attr, pairs, W_emb, b_emb, Wl0, bl0, Wr0, Wl1, bl1, Wr1, W1, b1, W2, b2):
    raise NotImplementedError("write your pallas kernel here")


# speedup vs baseline: 5.5903x; 5.5903x over previous
import functools

import jax
import jax.numpy as jnp
from jax import lax
from jax.experimental import pallas as pl
from jax.experimental.pallas import tpu as pltpu
from jax.experimental.pallas import tpu_sc as plsc

N = 10000
NPAD = 10240
E = 320000
H = 128
P = 8192

NC = 2
NS = 16
NW = NC * NS
CH = 128
NCHUNK = E // CH
CHUNK_BASE = NCHUNK // NW
CHUNK_REM = NCHUNK % NW
ROWS_PER_SUB = NPAD // NS
PAIRS_PER_W = P // NW
F32 = jnp.float32


def _mm_t(a, w):
    return lax.dot_general(a, w, (((1,), (1,)), ((), ())),
                           preferred_element_type=F32)


def _leaky(v):
    return jnp.where(v > 0, v, 0.1 * v)


def _tc1_body(x_ref, wemb_ref, bemb_ref, wl0_ref, wr0_ref, bl0_ref,
              m0_ref, r0_ref):
    h0 = _mm_t(x_ref[...], wemb_ref[...]) + bemb_ref[...]
    m0_ref[...] = _mm_t(h0, wl0_ref[...])
    r0_ref[...] = _mm_t(h0, wr0_ref[...]) + bl0_ref[...]


def _tc2_body(sp_ref, dg_ref, r0_ref, wl1_ref, wr1_ref, bl1_ref,
              m1_ref, r1_ref, inv_ref):
    s = sp_ref[pl.ds(0, N), :] + sp_ref[pl.ds(NPAD, N), :]
    deg = dg_ref[pl.ds(0, N), 0:1] + dg_ref[pl.ds(NPAD, N), 0:1]
    inv = 1.0 / jnp.maximum(deg, 1.0)
    h1 = _leaky(s * inv + r0_ref[...])
    m1_ref[...] = _mm_t(h1, wl1_ref[...])
    r1_ref[...] = _mm_t(h1, wr1_ref[...]) + bl1_ref[...]
    inv_ref[...] = inv


def _tc3_body(sp_ref, inv_ref, r1_ref, h2_ref):
    s = sp_ref[pl.ds(0, N), :] + sp_ref[pl.ds(NPAD, N), :]
    h2_ref[...] = s * inv_ref[...] + r1_ref[...]


def _tc4_body(g0_ref, g1_ref, w1a_ref, w1b_ref, b1_ref, w2_ref, b2_ref,
              out_ref):
    t = _mm_t(g0_ref[...], w1a_ref[...]) + _mm_t(g1_ref[...], w1b_ref[...])
    t = _leaky(t + b1_ref[...])
    out_ref[...] = jnp.sum(t * w2_ref[...], axis=1, keepdims=True) + b2_ref[...]


_MESH = plsc.VectorSubcoreMesh(core_axis_name="c", subcore_axis_name="s",
                               num_cores=NC, num_subcores=NS)


def _scatter_body(m_hbm, src_hbm, dst_hbm, z128_hbm,
                  sp_hbm, acc, sidx, didx, rows, sem):
    c = lax.axis_index("c")
    s = lax.axis_index("s")
    t = c * NS + s
    nbase = s * ROWS_PER_SUB
    pltpu.sync_copy(z128_hbm, acc.at[pl.ds(nbase, ROWS_PER_SUB)])
    plsc.subcore_barrier()

    start = t * CHUNK_BASE + jnp.minimum(t, CHUNK_REM)
    cnt = jnp.where(t < CHUNK_REM, CHUNK_BASE + 1, CHUNK_BASE)

    def step(k, carry):
        off = (start + k) * CH
        pltpu.sync_copy(src_hbm.at[pl.ds(off, CH)], sidx)
        pltpu.sync_copy(dst_hbm.at[pl.ds(off, CH)], didx)
        pltpu.async_copy(m_hbm.at[sidx], rows, sem).wait()
        pltpu.sync_copy(rows, acc.at[didx], add=True)
        return carry

    lax.fori_loop(0, cnt, step, 0)
    plsc.subcore_barrier()
    obase = c * NPAD + nbase
    pltpu.sync_copy(acc.at[pl.ds(nbase, ROWS_PER_SUB)],
                    sp_hbm.at[pl.ds(obase, ROWS_PER_SUB)])


def _deg_body(dst_hbm, z128_hbm, ones_hbm, dg_hbm, acc, didx, ones_v, sem):
    del sem
    c = lax.axis_index("c")
    s = lax.axis_index("s")
    t = c * NS + s
    nbase = s * ROWS_PER_SUB
    pltpu.sync_copy(z128_hbm, acc.at[pl.ds(nbase, ROWS_PER_SUB)])
    pltpu.sync_copy(ones_hbm, ones_v)
    plsc.subcore_barrier()

    start = t * CHUNK_BASE + jnp.minimum(t, CHUNK_REM)
    cnt = jnp.where(t < CHUNK_REM, CHUNK_BASE + 1, CHUNK_BASE)

    def step(k, carry):
        off = (start + k) * CH
        pltpu.sync_copy(dst_hbm.at[pl.ds(off, CH)], didx)
        pltpu.sync_copy(ones_v, acc.at[didx], add=True)
        return carry

    lax.fori_loop(0, cnt, step, 0)
    plsc.subcore_barrier()
    obase = c * NPAD + nbase
    pltpu.sync_copy(acc.at[pl.ds(nbase, ROWS_PER_SUB)],
                    dg_hbm.at[pl.ds(obase, ROWS_PER_SUB)])


def _pair_gather_body(h2_hbm, p0_hbm, p1_hbm, g0_hbm, g1_hbm,
                      idx0, idx1, rows0, rows1, sem0, sem1):
    c = lax.axis_index("c")
    s = lax.axis_index("s")
    t = c * NS + s
    base = t * PAIRS_PER_W
    for k in range(PAIRS_PER_W // CH):
        off = base + k * CH
        pltpu.sync_copy(p0_hbm.at[pl.ds(off, CH)], idx0)
        pltpu.sync_copy(p1_hbm.at[pl.ds(off, CH)], idx1)
        cp0 = pltpu.async_copy(h2_hbm.at[idx0], rows0, sem0)
        cp1 = pltpu.async_copy(h2_hbm.at[idx1], rows1, sem1)
        cp0.wait()
        pltpu.sync_copy(rows0, g0_hbm.at[pl.ds(off, CH)])
        cp1.wait()
        pltpu.sync_copy(rows1, g1_hbm.at[pl.ds(off, CH)])


_conv_scatter = pl.kernel(
    _scatter_body,
    out_type=jax.ShapeDtypeStruct((NC * NPAD, H), F32),
    mesh=_MESH,
    scratch_types=[
        pltpu.VMEM_SHARED((NPAD, H), F32),
        pltpu.VMEM((CH,), jnp.int32),
        pltpu.VMEM((CH,), jnp.int32),
        pltpu.VMEM((CH, H), F32),
        pltpu.SemaphoreType.DMA,
    ],
)

_deg_scatter = pl.kernel(
    _deg_body,
    out_type=jax.ShapeDtypeStruct((NC * NPAD, H), F32),
    mesh=_MESH,
    scratch_types=[
        pltpu.VMEM_SHARED((NPAD, H), F32),
        pltpu.VMEM((CH,), jnp.int32),
        pltpu.VMEM((CH, H), F32),
        pltpu.SemaphoreType.DMA,
    ],
)

_pair_gather = pl.kernel(
    _pair_gather_body,
    out_type=(jax.ShapeDtypeStruct((P, H), F32),
              jax.ShapeDtypeStruct((P, H), F32)),
    mesh=_MESH,
    scratch_types=[
        pltpu.VMEM((CH,), jnp.int32),
        pltpu.VMEM((CH,), jnp.int32),
        pltpu.VMEM((CH, H), F32),
        pltpu.VMEM((CH, H), F32),
        pltpu.SemaphoreType.DMA,
        pltpu.SemaphoreType.DMA,
    ],
)


def _tc_call(body, out_shapes, *args):
    return pl.pallas_call(body, out_shape=out_shapes)(*args)


def kernel(x, edge_index, edge_attr, pairs, W_emb, b_emb, Wl0, bl0, Wr0,
           Wl1, bl1, Wr1, W1, b1, W2, b2):
    del edge_attr
    src = edge_index[0]
    dst = edge_index[1]
    pt = pairs.T
    p0 = jnp.asarray(pt[0])
    p1 = jnp.asarray(pt[1])
    z128 = jnp.zeros((ROWS_PER_SUB, H), F32)
    ones128 = jnp.ones((CH, H), F32)
    sds = jax.ShapeDtypeStruct

    dg = _deg_scatter(dst, z128, ones128)
    m0, r0 = _tc_call(_tc1_body, (sds((N, H), F32), sds((N, H), F32)),
                      x, W_emb, b_emb.reshape(1, H), Wl0, Wr0,
                      bl0.reshape(1, H))
    sp0 = _conv_scatter(m0, src, dst, z128)
    m1, r1, inv = _tc_call(
        _tc2_body, (sds((N, H), F32), sds((N, H), F32), sds((N, 1), F32)),
        sp0, dg, r0, Wl1, Wr1, bl1.reshape(1, H))
    sp1 = _conv_scatter(m1, src, dst, z128)
    h2 = _tc_call(_tc3_body, sds((N, H), F32), sp1, inv, r1)
    g0, g1 = _pair_gather(h2, p0, p1)
    out = _tc_call(_tc4_body, sds((P, 1), F32),
                   g0, g1, W1[:, :H], W1[:, H:], b1.reshape(1, H),
                   W2, jnp.broadcast_to(b2.reshape(1, 1), (P, 1)))
    return out


# speedup vs baseline: 9.8991x; 1.7708x over previous
import functools

import jax
import jax.numpy as jnp
from jax import lax
from jax.experimental import pallas as pl
from jax.experimental.pallas import tpu as pltpu
from jax.experimental.pallas import tpu_sc as plsc

N = 10000
NPAD = 10240
E = 320000
H = 128
P = 8192

NC = 2
NS = 16
NW = NC * NS
CH = 128
CNT = 80
NPH = 2
PH = CNT // NPH
NCH_PAD = NW * CNT
EPAD = NCH_PAD * CH
ROWS_PER_SUB = NPAD // NS
PAIRS_PER_W = P // NW
F32 = jnp.float32


def _mm_t(a, w):
    return lax.dot_general(a, w, (((1,), (1,)), ((), ())),
                           preferred_element_type=F32)


def _leaky(v):
    return jnp.where(v > 0, v, 0.1 * v)


def _tc1_body(x_ref, wemb_ref, bemb_ref, wl0_ref, wr0_ref, bl0_ref,
              m0_ref, r0_ref):
    h0 = _mm_t(x_ref[...], wemb_ref[...]) + bemb_ref[...]
    m0_ref[...] = _mm_t(h0, wl0_ref[...])
    r0_ref[...] = _mm_t(h0, wr0_ref[...]) + bl0_ref[...]


def _tc2_body(sp_ref, dg_ref, r0_ref, wl1_ref, wr1_ref, bl1_ref,
              m1_ref, r1_ref, inv_ref):
    s = sp_ref[pl.ds(0, N), :] + sp_ref[pl.ds(NPAD, N), :]
    deg = dg_ref[pl.ds(0, N), 0:1] + dg_ref[pl.ds(NPAD, N), 0:1]
    inv = 1.0 / jnp.maximum(deg, 1.0)
    h1 = _leaky(s * inv + r0_ref[...])
    m1_ref[...] = _mm_t(h1, wl1_ref[...])
    r1_ref[...] = _mm_t(h1, wr1_ref[...]) + bl1_ref[...]
    inv_ref[...] = inv


def _tc3_body(sp_ref, inv_ref, r1_ref, h2_ref):
    s = sp_ref[pl.ds(0, N), :] + sp_ref[pl.ds(NPAD, N), :]
    h2_ref[...] = s * inv_ref[...] + r1_ref[...]


def _tc4_body(g0_ref, g1_ref, w1a_ref, w1b_ref, b1_ref, w2_ref, b2_ref,
              out_ref):
    t = _mm_t(g0_ref[...], w1a_ref[...]) + _mm_t(g1_ref[...], w1b_ref[...])
    t = _leaky(t + b1_ref[...])
    out_ref[...] = jnp.sum(t * w2_ref[...], axis=1, keepdims=True) + b2_ref[...]


_MESH = plsc.VectorSubcoreMesh(core_axis_name="c", subcore_axis_name="s",
                               num_cores=NC, num_subcores=NS)


def _scatter_body(m_hbm, src2_hbm, dst2_hbm, z128_hbm,
                  sp_hbm, acc, sidx, didx, rows0, rows1, gsem0, gsem1):
    c = lax.axis_index("c")
    s = lax.axis_index("s")
    t = c * NS + s
    nbase = s * ROWS_PER_SUB
    pltpu.sync_copy(z128_hbm, acc.at[pl.ds(nbase, ROWS_PER_SUB)])
    plsc.subcore_barrier()

    for p in range(NPH):
        cb = t * CNT + p * PH
        pltpu.sync_copy(src2_hbm.at[pl.ds(cb, PH)], sidx)
        pltpu.sync_copy(dst2_hbm.at[pl.ds(cb, PH)], didx)
        pltpu.async_copy(m_hbm.at[sidx.at[0]], rows0, gsem0)

        def step(i, carry):
            k0 = 2 * i
            k1 = k0 + 1
            pltpu.async_copy(m_hbm.at[sidx.at[k1]], rows1, gsem1)
            pltpu.make_async_copy(m_hbm.at[sidx.at[k0]], rows0, gsem0).wait()
            pltpu.sync_copy(rows0, acc.at[didx.at[k0]], add=True)
            k2 = jnp.where(k0 + 2 < PH, k0 + 2, 0)
            pltpu.async_copy(m_hbm.at[sidx.at[k2]], rows0, gsem0)
            pltpu.make_async_copy(m_hbm.at[sidx.at[k1]], rows1, gsem1).wait()
            pltpu.sync_copy(rows1, acc.at[didx.at[k1]], add=True)
            return carry

        lax.fori_loop(0, PH // 2, step, 0)
        pltpu.make_async_copy(m_hbm.at[sidx.at[0]], rows0, gsem0).wait()
    plsc.subcore_barrier()
    obase = c * NPAD + nbase
    pltpu.sync_copy(acc.at[pl.ds(nbase, ROWS_PER_SUB)],
                    sp_hbm.at[pl.ds(obase, ROWS_PER_SUB)])


def _deg_body(dst2_hbm, z128_hbm, ones_hbm, dg_hbm, acc, didx, ones_v, sem):
    del sem
    c = lax.axis_index("c")
    s = lax.axis_index("s")
    t = c * NS + s
    nbase = s * ROWS_PER_SUB
    pltpu.sync_copy(z128_hbm, acc.at[pl.ds(nbase, ROWS_PER_SUB)])
    pltpu.sync_copy(ones_hbm, ones_v)
    cb = t * CNT
    pltpu.sync_copy(dst2_hbm.at[pl.ds(cb, CNT)], didx)
    plsc.subcore_barrier()

    def step(k, carry):
        pltpu.sync_copy(ones_v, acc.at[didx.at[k]], add=True)
        return carry

    lax.fori_loop(0, CNT, step, 0)
    plsc.subcore_barrier()
    obase = c * NPAD + nbase
    pltpu.sync_copy(acc.at[pl.ds(nbase, ROWS_PER_SUB)],
                    dg_hbm.at[pl.ds(obase, ROWS_PER_SUB)])


def _pair_gather_body(h2_hbm, p0_hbm, p1_hbm, g0_hbm, g1_hbm,
                      idx0, idx1, rows0, rows1, sem0, sem1):
    c = lax.axis_index("c")
    s = lax.axis_index("s")
    t = c * NS + s
    base = t * PAIRS_PER_W
    for k in range(PAIRS_PER_W // CH):
        off = base + k * CH
        pltpu.sync_copy(p0_hbm.at[pl.ds(off, CH)], idx0)
        pltpu.sync_copy(p1_hbm.at[pl.ds(off, CH)], idx1)
        cp0 = pltpu.async_copy(h2_hbm.at[idx0], rows0, sem0)
        cp1 = pltpu.async_copy(h2_hbm.at[idx1], rows1, sem1)
        cp0.wait()
        pltpu.sync_copy(rows0, g0_hbm.at[pl.ds(off, CH)])
        cp1.wait()
        pltpu.sync_copy(rows1, g1_hbm.at[pl.ds(off, CH)])


_conv_scatter = pl.kernel(
    _scatter_body,
    out_type=jax.ShapeDtypeStruct((NC * NPAD, H), F32),
    mesh=_MESH,
    scratch_types=[
        pltpu.VMEM_SHARED((NPAD, H), F32),
        pltpu.VMEM((PH, CH), jnp.int32),
        pltpu.VMEM((PH, CH), jnp.int32),
        pltpu.VMEM((CH, H), F32),
        pltpu.VMEM((CH, H), F32),
        pltpu.SemaphoreType.DMA,
        pltpu.SemaphoreType.DMA,
    ],
)

_deg_scatter = pl.kernel(
    _deg_body,
    out_type=jax.ShapeDtypeStruct((NC * NPAD, H), F32),
    mesh=_MESH,
    scratch_types=[
        pltpu.VMEM_SHARED((NPAD, H), F32),
        pltpu.VMEM((CNT, CH), jnp.int32),
        pltpu.VMEM((CH, H), F32),
        pltpu.SemaphoreType.DMA,
    ],
)

_pair_gather = pl.kernel(
    _pair_gather_body,
    out_type=(jax.ShapeDtypeStruct((P, H), F32),
              jax.ShapeDtypeStruct((P, H), F32)),
    mesh=_MESH,
    scratch_types=[
        pltpu.VMEM((CH,), jnp.int32),
        pltpu.VMEM((CH,), jnp.int32),
        pltpu.VMEM((CH, H), F32),
        pltpu.VMEM((CH, H), F32),
        pltpu.SemaphoreType.DMA,
        pltpu.SemaphoreType.DMA,
    ],
)


def _tc_call(body, out_shapes, *args):
    return pl.pallas_call(body, out_shape=out_shapes)(*args)


def kernel(x, edge_index, edge_attr, pairs, W_emb, b_emb, Wl0, bl0, Wr0,
           Wl1, bl1, Wr1, W1, b1, W2, b2):
    del edge_attr
    npadE = EPAD - E
    pad_src = jnp.arange(npadE, dtype=jnp.int32) % N
    pad_dst = N + jnp.arange(npadE, dtype=jnp.int32) % (NPAD - N)
    src2 = jnp.concatenate([edge_index[0], pad_src]).reshape(NCH_PAD, CH)
    dst2 = jnp.concatenate([edge_index[1], pad_dst]).reshape(NCH_PAD, CH)
    pt = pairs.T
    p0 = jnp.asarray(pt[0])
    p1 = jnp.asarray(pt[1])
    z128 = jnp.zeros((ROWS_PER_SUB, H), F32)
    ones128 = jnp.ones((CH, H), F32)
    sds = jax.ShapeDtypeStruct

    dg = _deg_scatter(dst2, z128, ones128)
    m0, r0 = _tc_call(_tc1_body, (sds((N, H), F32), sds((N, H), F32)),
                      x, W_emb, b_emb.reshape(1, H), Wl0, Wr0,
                      bl0.reshape(1, H))
    sp0 = _conv_scatter(m0, src2, dst2, z128)
    m1, r1, inv = _tc_call(
        _tc2_body, (sds((N, H), F32), sds((N, H), F32), sds((N, 1), F32)),
        sp0, dg, r0, Wl1, Wr1, bl1.reshape(1, H))
    sp1 = _conv_scatter(m1, src2, dst2, z128)
    h2 = _tc_call(_tc3_body, sds((N, H), F32), sp1, inv, r1)
    g0, g1 = _pair_gather(h2, p0, p1)
    out = _tc_call(_tc4_body, sds((P, 1), F32),
                   g0, g1, W1[:, :H], W1[:, H:], b1.reshape(1, H),
                   W2, jnp.broadcast_to(b2.reshape(1, 1), (P, 1)))
    return out


# speedup vs baseline: 11.9394x; 1.2061x over previous
import functools

import jax
import jax.numpy as jnp
from jax import lax
from jax.experimental import pallas as pl
from jax.experimental.pallas import tpu as pltpu
from jax.experimental.pallas import tpu_sc as plsc

N = 10000
NPAD = 10240
E = 320000
H = 128
P = 8192

NC = 2
NS = 16
NW = NC * NS
CH = 128
CNT = 80
NPH = 2
PH = CNT // NPH
NCH_PAD = NW * CNT
EPAD = NCH_PAD * CH
ROWS_PER_SUB = NPAD // NS
PAIRS_PER_W = P // NW
F32 = jnp.float32


def _mm_t(a, w):
    return lax.dot_general(a, w, (((1,), (1,)), ((), ())),
                           preferred_element_type=F32)


def _leaky(v):
    return jnp.where(v > 0, v, 0.1 * v)


def _tc1_body(x_ref, wemb_ref, bemb_ref, wl0_ref, wr0_ref, bl0_ref,
              m0_ref, r0_ref):
    h0 = _mm_t(x_ref[...], wemb_ref[...]) + bemb_ref[...]
    m0_ref[...] = _mm_t(h0, wl0_ref[...])
    r0_ref[...] = _mm_t(h0, wr0_ref[...]) + bl0_ref[...]


def _tc2_body(sp_ref, dg_ref, r0_ref, wl1_ref, wr1_ref, bl1_ref,
              m1_ref, r1_ref, inv_ref):
    s = sp_ref[pl.ds(0, N), :] + sp_ref[pl.ds(NPAD, N), :]
    deg = dg_ref[pl.ds(0, N), :] + dg_ref[pl.ds(NPAD, N), :]
    inv = 1.0 / jnp.maximum(deg, 1.0)
    h1 = _leaky(s * inv + r0_ref[...])
    m1_ref[...] = _mm_t(h1, wl1_ref[...])
    r1_ref[...] = _mm_t(h1, wr1_ref[...]) + bl1_ref[...]
    inv_ref[...] = inv


def _tc3_body(sp_ref, inv_ref, r1_ref, h2_ref):
    s = sp_ref[pl.ds(0, N), :] + sp_ref[pl.ds(NPAD, N), :]
    h2_ref[...] = s * inv_ref[...] + r1_ref[...]


def _tc4_body(g0_ref, g1_ref, w1a_ref, w1b_ref, b1_ref, w2_ref, b2_ref,
              out_ref):
    t = _mm_t(g0_ref[...], w1a_ref[...]) + _mm_t(g1_ref[...], w1b_ref[...])
    t = _leaky(t + b1_ref[...])
    out_ref[...] = jnp.sum(t * w2_ref[...], axis=1, keepdims=True) + b2_ref[...]


_MESH = plsc.VectorSubcoreMesh(core_axis_name="c", subcore_axis_name="s",
                               num_cores=NC, num_subcores=NS)


def _scatter_body(m_hbm, src2_hbm, dst2_hbm, z128_hbm,
                  sp_hbm, acc, sidx, didx, rows0, rows1, gsem0, gsem1):
    c = lax.axis_index("c")
    s = lax.axis_index("s")
    t = c * NS + s
    nbase = s * ROWS_PER_SUB
    pltpu.sync_copy(z128_hbm, acc.at[pl.ds(nbase, ROWS_PER_SUB)])
    plsc.subcore_barrier()

    for p in range(NPH):
        cb = t * CNT + p * PH
        pltpu.sync_copy(src2_hbm.at[pl.ds(cb, PH)], sidx)
        pltpu.sync_copy(dst2_hbm.at[pl.ds(cb, PH)], didx)
        pltpu.async_copy(m_hbm.at[sidx.at[0]], rows0, gsem0)

        def step(i, carry):
            k0 = 2 * i
            k1 = k0 + 1
            pltpu.async_copy(m_hbm.at[sidx.at[k1]], rows1, gsem1)
            pltpu.make_async_copy(m_hbm.at[sidx.at[k0]], rows0, gsem0).wait()
            pltpu.sync_copy(rows0, acc.at[didx.at[k0]], add=True)
            k2 = jnp.where(k0 + 2 < PH, k0 + 2, 0)
            pltpu.async_copy(m_hbm.at[sidx.at[k2]], rows0, gsem0)
            pltpu.make_async_copy(m_hbm.at[sidx.at[k1]], rows1, gsem1).wait()
            pltpu.sync_copy(rows1, acc.at[didx.at[k1]], add=True)
            return carry

        lax.fori_loop(0, PH // 2, step, 0)
        pltpu.make_async_copy(m_hbm.at[sidx.at[0]], rows0, gsem0).wait()
    plsc.subcore_barrier()
    obase = c * NPAD + nbase
    pltpu.sync_copy(acc.at[pl.ds(nbase, ROWS_PER_SUB)],
                    sp_hbm.at[pl.ds(obase, ROWS_PER_SUB)])


def _deg_body(dst1_hbm, z1d_hbm, dg_hbm, shared, didx, dcnt, buf, res):
    c = lax.axis_index("c")
    s = lax.axis_index("s")
    t = c * NS + s
    pltpu.sync_copy(z1d_hbm, dcnt)
    eb = t * CNT * CH
    pltpu.sync_copy(dst1_hbm.at[pl.ds(eb, CNT * CH)], didx)
    ones_v = jnp.ones((16,), F32)

    def step(k, carry):
        for j in range(CH // 16):
            idxv = didx[pl.ds(k * CH + j * 16, 16)]
            plsc.addupdate_scatter(dcnt, [idxv], ones_v)
        return carry

    lax.fori_loop(0, CNT, step, 0)
    pltpu.sync_copy(dcnt, shared.at[s])
    plsc.subcore_barrier()
    nbase = s * ROWS_PER_SUB
    for q in range(NS):
        pltpu.sync_copy(shared.at[q, pl.ds(nbase, ROWS_PER_SUB)],
                        buf.at[pl.ds(q * ROWS_PER_SUB, ROWS_PER_SUB)])

    def red(r, carry):
        v = buf[pl.ds(r * 16, 16)]
        for q in range(1, NS):
            v = v + buf[pl.ds(q * ROWS_PER_SUB + r * 16, 16)]
        res[pl.ds(r * 16, 16)] = v
        return carry

    lax.fori_loop(0, ROWS_PER_SUB // 16, red, 0)
    pltpu.sync_copy(res, dg_hbm.at[pl.ds(c * NPAD + nbase, ROWS_PER_SUB)])


def _pair_gather_body(h2_hbm, p0_hbm, p1_hbm, g0_hbm, g1_hbm,
                      idx0, idx1, rows0, rows1, sem0, sem1):
    c = lax.axis_index("c")
    s = lax.axis_index("s")
    t = c * NS + s
    base = t * PAIRS_PER_W
    for k in range(PAIRS_PER_W // CH):
        off = base + k * CH
        pltpu.sync_copy(p0_hbm.at[pl.ds(off, CH)], idx0)
        pltpu.sync_copy(p1_hbm.at[pl.ds(off, CH)], idx1)
        cp0 = pltpu.async_copy(h2_hbm.at[idx0], rows0, sem0)
        cp1 = pltpu.async_copy(h2_hbm.at[idx1], rows1, sem1)
        cp0.wait()
        pltpu.sync_copy(rows0, g0_hbm.at[pl.ds(off, CH)])
        cp1.wait()
        pltpu.sync_copy(rows1, g1_hbm.at[pl.ds(off, CH)])


_conv_scatter = pl.kernel(
    _scatter_body,
    out_type=jax.ShapeDtypeStruct((NC * NPAD, H), F32),
    mesh=_MESH,
    scratch_types=[
        pltpu.VMEM_SHARED((NPAD, H), F32),
        pltpu.VMEM((PH, CH), jnp.int32),
        pltpu.VMEM((PH, CH), jnp.int32),
        pltpu.VMEM((CH, H), F32),
        pltpu.VMEM((CH, H), F32),
        pltpu.SemaphoreType.DMA,
        pltpu.SemaphoreType.DMA,
    ],
)

_deg_scatter = pl.kernel(
    _deg_body,
    out_type=jax.ShapeDtypeStruct((NC * NPAD,), F32),
    mesh=_MESH,
    scratch_types=[
        pltpu.VMEM_SHARED((NS, NPAD), F32),
        pltpu.VMEM((CNT * CH,), jnp.int32),
        pltpu.VMEM((NPAD,), F32),
        pltpu.VMEM((NS * ROWS_PER_SUB,), F32),
        pltpu.VMEM((ROWS_PER_SUB,), F32),
    ],
    compiler_params=pltpu.CompilerParams(needs_layout_passes=False),
)

_pair_gather = pl.kernel(
    _pair_gather_body,
    out_type=(jax.ShapeDtypeStruct((P, H), F32),
              jax.ShapeDtypeStruct((P, H), F32)),
    mesh=_MESH,
    scratch_types=[
        pltpu.VMEM((CH,), jnp.int32),
        pltpu.VMEM((CH,), jnp.int32),
        pltpu.VMEM((CH, H), F32),
        pltpu.VMEM((CH, H), F32),
        pltpu.SemaphoreType.DMA,
        pltpu.SemaphoreType.DMA,
    ],
)


def _tc_call(body, out_shapes, *args):
    return pl.pallas_call(body, out_shape=out_shapes)(*args)


def kernel(x, edge_index, edge_attr, pairs, W_emb, b_emb, Wl0, bl0, Wr0,
           Wl1, bl1, Wr1, W1, b1, W2, b2):
    del edge_attr
    npadE = EPAD - E
    pad_src = jnp.arange(npadE, dtype=jnp.int32) % N
    pad_dst = N + jnp.arange(npadE, dtype=jnp.int32) % (NPAD - N)
    src2 = jnp.concatenate([edge_index[0], pad_src]).reshape(NCH_PAD, CH)
    dst2 = jnp.concatenate([edge_index[1], pad_dst]).reshape(NCH_PAD, CH)
    pt = pairs.T
    p0 = jnp.asarray(pt[0])
    p1 = jnp.asarray(pt[1])
    z128 = jnp.zeros((ROWS_PER_SUB, H), F32)
    z1d = jnp.zeros((NPAD,), F32)
    sds = jax.ShapeDtypeStruct

    dg = _deg_scatter(dst2.reshape(-1), z1d).reshape(2 * NPAD, 1)
    m0, r0 = _tc_call(_tc1_body, (sds((N, H), F32), sds((N, H), F32)),
                      x, W_emb, b_emb.reshape(1, H), Wl0, Wr0,
                      bl0.reshape(1, H))
    sp0 = _conv_scatter(m0, src2, dst2, z128)
    m1, r1, inv = _tc_call(
        _tc2_body, (sds((N, H), F32), sds((N, H), F32), sds((N, 1), F32)),
        sp0, dg, r0, Wl1, Wr1, bl1.reshape(1, H))
    sp1 = _conv_scatter(m1, src2, dst2, z128)
    h2 = _tc_call(_tc3_body, sds((N, H), F32), sp1, inv, r1)
    g0, g1 = _pair_gather(h2, p0, p1)
    out = _tc_call(_tc4_body, sds((P, 1), F32),
                   g0, g1, W1[:, :H], W1[:, H:], b1.reshape(1, H),
                   W2, jnp.broadcast_to(b2.reshape(1, 1), (P, 1)))
    return out
